# Initial kernel scaffold; baseline (speedup 1.0000x reference)
import jax
import jax.numpy as jnp
from jax.experimental import pallas as pl


def kernel(edge_index, vertex_features, batch, params):
    raise NotImplementedError("write your pallas kernel here")


# speedup vs baseline: 2.6192x; 2.6192x over previous
import functools

import jax
import jax.numpy as jnp
from jax import lax
from jax.experimental import pallas as pl
from jax.experimental.pallas import tpu as pltpu
from jax.experimental.pallas import tpu_sc as plsc

N = 10000
E = 320000
NUM_GRAPHS = 16
LAYER_DIMS = [(128, 128), (128, 256), (256, 256)]

NC = 2
NS = 16
NW = NC * NS
NP = 10240
R = NP // NW
G = 64
C = 4000
S = 8192
F = 4096
GMAX = 128
CAP = E + GMAX

_mesh = plsc.VectorSubcoreMesh(core_axis_name="c", subcore_axis_name="s")


def _wid():
    return lax.axis_index("s") * NC + lax.axis_index("c")


PK = 14
PKM = (1 << PK) - 1


@functools.partial(
    pl.kernel,
    out_type=(
        jax.ShapeDtypeStruct((NW * CAP,), jnp.int32),
        jax.ShapeDtypeStruct((NW * 16,), jnp.int32),
    ),
    mesh=_mesh,
    scratch_types=[
        pltpu.VMEM((C,), jnp.int32),
        pltpu.VMEM((C,), jnp.int32),
        pltpu.VMEM((S,), jnp.int32),
        pltpu.VMEM((16,), jnp.int32),
    ],
)
def _filter_edges(src_hbm, dst_hbm, sel_hbm, counts_hbm,
                  srcc, dstc, stg, cntv):
    wid = _wid()
    lo = wid * R
    lane = lax.iota(jnp.int32, 16)

    def chunk_body(ci, carry):
        cur, ocur = carry
        pltpu.sync_copy(src_hbm.at[pl.ds(ci * C, C)], srcc)
        pltpu.sync_copy(dst_hbm.at[pl.ds(ci * C, C)], dstc)

        def grp(gi, cur2):
            d = dstc[pl.ds(gi * 16, 16)]
            s = srcc[pl.ds(gi * 16, 16)]
            ld = d - lo
            msk = (ld >= 0) & (ld < R)
            mi = jnp.where(msk, 1, 0)
            v = (ld << PK) + s
            c = jnp.zeros((16,), jnp.int32)
            lcnt = jnp.int32(0)
            for l in range(16):
                mil = mi[l]
                pos = jnp.where(mil > 0, lcnt, 31)
                c = jnp.where(lane == pos, jnp.broadcast_to(v[l], (16,)), c)
                lcnt = lcnt + mil
            stg[pl.ds(cur2, 16)] = c
            return cur2 + lcnt

        cur = lax.fori_loop(0, C // 16, grp, cur)

        def do_flush(args):
            cur3, ocur3 = args
            pltpu.sync_copy(stg.at[pl.ds(0, F)],
                            sel_hbm.at[pl.ds(pl.multiple_of(wid * CAP + ocur3, 64), F)])
            tail = cur3 - F

            def mv(mi_, _):
                stg[pl.ds(mi_ * 16, 16)] = stg[pl.ds(F + mi_ * 16, 16)]
                return 0

            lax.fori_loop(0, (tail + 15) // 16, mv, 0)
            return tail, ocur3 + F

        return lax.cond(cur >= F, do_flush, lambda a: a, (cur, ocur))

    cur, ocur = lax.fori_loop(0, E // C, chunk_body,
                              (jnp.int32(0), jnp.int32(0)))

    gv = jnp.full((16,), R << PK, jnp.int32)
    for t in range(GMAX // 16):
        stg[pl.ds(cur + t * 16, 16)] = gv
    cur = cur + lax.rem(GMAX - lax.rem(cur, GMAX), GMAX)

    def drain(j, _):
        pltpu.sync_copy(stg.at[pl.ds(j * G, G)],
                        sel_hbm.at[pl.ds(pl.multiple_of(wid * CAP + ocur + j * G, 64), G)])
        return 0

    lax.fori_loop(0, cur // G, drain, 0)
    cntv[pl.ds(0, 16)] = jnp.broadcast_to(ocur + cur, (16,)).astype(jnp.int32)
    pltpu.sync_copy(cntv, counts_hbm.at[pl.ds(pl.multiple_of(wid * 16, 16), 16)])


def _make_segmax(dout, gb):
    @functools.partial(
        pl.kernel,
        out_type=jax.ShapeDtypeStruct((NP, dout), jnp.float32),
        mesh=_mesh,
        scratch_types=[
            pltpu.VMEM((R + 1, dout), jnp.float32),
            pltpu.VMEM((gb,), jnp.int32),
            pltpu.VMEM((gb,), jnp.int32),
            pltpu.VMEM((gb,), jnp.int32),
            pltpu.VMEM((gb,), jnp.int32),
            pltpu.VMEM((gb,), jnp.int32),
            pltpu.VMEM((gb,), jnp.int32),
            pltpu.VMEM((gb, dout), jnp.float32),
            pltpu.VMEM((gb, dout), jnp.float32),
            pltpu.VMEM((16,), jnp.int32),
            pltpu.SemaphoreType.DMA,
            pltpu.SemaphoreType.DMA,
        ],
    )
    def seg(y_hbm, sel_hbm, counts_hbm, m_hbm,
            acc, pk0, is0, il0, pk1, is1, il1, rows0, rows1, cntv,
            sem0, sem1):
        wid = _wid()
        ninf = jnp.full((16,), -jnp.inf, jnp.float32)

        def init_r(r, _):
            for j in range(dout // 16):
                acc[r, pl.ds(j * 16, 16)] = ninf
            return 0

        lax.fori_loop(0, R + 1, init_r, 0)
        pltpu.sync_copy(counts_hbm.at[pl.ds(pl.multiple_of(wid * 16, 16), 16)], cntv)
        total = cntv[pl.ds(0, 16)][0]
        nb = total // gb

        def fetch(b, pkv, isv, ilv):
            pltpu.sync_copy(
                sel_hbm.at[pl.ds(pl.multiple_of(wid * CAP + b * gb, 64), gb)],
                pkv)
            for q in range(gb // 16):
                w = pkv[pl.ds(q * 16, 16)]
                isv[pl.ds(q * 16, 16)] = w & PKM
                ilv[pl.ds(q * 16, 16)] = lax.shift_right_logical(w, PK)

        def start(isv, rows, sem):
            pltpu.make_async_copy(y_hbm.at[isv], rows, sem).start()

        def wait(isv, rows, sem):
            pltpu.make_async_copy(y_hbm.at[isv], rows, sem).wait()

        def accum(ilv, rows):
            def edge16(e, _):
                rv = ilv[pl.ds(e * 16, 16)]
                for l in range(16):
                    r = rv[l]
                    for j in range(dout // 16):
                        sl = pl.ds(j * 16, 16)
                        acc[r, sl] = jnp.maximum(acc[r, sl],
                                                 rows[e * 16 + l, sl])
                return 0

            lax.fori_loop(0, gb // 16, edge16, 0)

        @pl.when(nb > 0)
        def _():
            fetch(0, pk0, is0, il0)
            start(is0, rows0, sem0)

        def pair(pb, _):
            b1 = 2 * pb + 1
            b2 = 2 * pb + 2

            @pl.when(b1 < nb)
            def _():
                fetch(b1, pk1, is1, il1)
                start(is1, rows1, sem1)

            wait(is0, rows0, sem0)
            accum(il0, rows0)

            @pl.when(b1 < nb)
            def _():
                @pl.when(b2 < nb)
                def _():
                    fetch(b2, pk0, is0, il0)
                    start(is0, rows0, sem0)

                wait(is1, rows1, sem1)
                accum(il1, rows1)

            return 0

        lax.fori_loop(0, (nb + 1) // 2, pair, 0)
        pltpu.sync_copy(acc.at[pl.ds(0, R)], m_hbm.at[pl.ds(pl.multiple_of(wid * R, R), R)])

    return seg


_segmax = {128: _make_segmax(128, 128), 256: _make_segmax(256, 64)}


BM = 256
NBLK = NP // BM


def _mm_body(x_ref, w_ref, o_ref):
    o_ref[...] = lax.dot_general(
        x_ref[...], w_ref[...], (((1,), (1,)), ((), ())),
        preferred_element_type=jnp.float32)


def _mm(x, w):
    din = x.shape[1]
    dout = w.shape[0]
    return pl.pallas_call(
        _mm_body,
        grid=(NBLK,),
        in_specs=[pl.BlockSpec((BM, din), lambda i: (i, 0)),
                  pl.BlockSpec((dout, din), lambda i: (0, 0))],
        out_specs=pl.BlockSpec((BM, dout), lambda i: (i, 0)),
        out_shape=jax.ShapeDtypeStruct((NP, dout), jnp.float32),
    )(x, w)


def _post_body(x_ref, y_ref, m_ref, be_ref, w1x_ref, w1a_ref, b1_ref,
               g1_ref, bb1_ref, w2_ref, b2_ref, o_ref):
    m = m_ref[...]
    agg = jnp.where(jnp.isfinite(m), m - y_ref[...] + be_ref[...], 0.0)
    dn = (((1,), (1,)), ((), ()))
    h = (lax.dot_general(x_ref[...], w1x_ref[...], dn,
                         preferred_element_type=jnp.float32)
         + lax.dot_general(agg, w1a_ref[...], dn,
                           preferred_element_type=jnp.float32)
         + b1_ref[...])
    mu = jnp.mean(h, axis=-1, keepdims=True)
    var = jnp.mean((h - mu) ** 2, axis=-1, keepdims=True)
    h = (h - mu) / jnp.sqrt(var + 1e-5) * g1_ref[...] + bb1_ref[...]
    h = jnp.maximum(h, 0.0)
    o_ref[...] = lax.dot_general(h, w2_ref[...], dn,
                                 preferred_element_type=jnp.float32) + b2_ref[...]


def _post(x, y, m, p, din, dout):
    w1x = p["W1"][:, :din]
    w1a = p["W1"][:, din:]
    row = lambda a: a.reshape(1, -1)
    full = lambda shape: pl.BlockSpec(shape, lambda i: (0, 0))
    return pl.pallas_call(
        _post_body,
        grid=(NBLK,),
        in_specs=[
            pl.BlockSpec((BM, din), lambda i: (i, 0)),
            pl.BlockSpec((BM, dout), lambda i: (i, 0)),
            pl.BlockSpec((BM, dout), lambda i: (i, 0)),
            full((1, dout)), full((dout, din)), full((dout, dout)),
            full((1, dout)), full((1, dout)), full((1, dout)),
            full((dout, dout)), full((1, dout)),
        ],
        out_specs=pl.BlockSpec((BM, dout), lambda i: (i, 0)),
        out_shape=jax.ShapeDtypeStruct((NP, dout), jnp.float32),
    )(x, y, m, row(p["be"]), w1x, w1a, row(p["b1"]), row(p["g1"]),
      row(p["bb1"]), p["W2"], row(p["b2"]))


def _pool_body(x_ref, b_ref, o_ref):
    i = pl.program_id(0)
    xb = x_ref[...]
    bcol = b_ref[...].reshape(BM, 1)
    rows = [jnp.max(jnp.where(bcol == g, xb, -jnp.inf), axis=0)
            for g in range(NUM_GRAPHS)]
    res = jnp.stack(rows, axis=0)

    @pl.when(i == 0)
    def _():
        o_ref[...] = res

    @pl.when(i > 0)
    def _():
        o_ref[...] = jnp.maximum(o_ref[...], res)

    @pl.when(i == NBLK - 1)
    def _():
        o_ref[...] = jnp.where(jnp.isfinite(o_ref[...]), o_ref[...], 0.0)


def _pool(x, batch3):
    dout = x.shape[1]
    return pl.pallas_call(
        _pool_body,
        grid=(NBLK,),
        in_specs=[pl.BlockSpec((BM, dout), lambda i: (i, 0)),
                  pl.BlockSpec((1, 1, BM), lambda i: (i, 0, 0))],
        out_specs=pl.BlockSpec((NUM_GRAPHS, dout), lambda i: (0, 0)),
        out_shape=jax.ShapeDtypeStruct((NUM_GRAPHS, dout), jnp.float32),
    )(x, batch3)


def kernel(edge_index, vertex_features, batch, params):
    src = edge_index[0]
    dst = edge_index[1]
    x = jnp.zeros((NP, 128), jnp.float32).at[:N].set(vertex_features)
    batch3 = jnp.full((NP,), NUM_GRAPHS, jnp.int32).at[:N].set(batch)
    batch3 = batch3.reshape(NBLK, 1, BM)

    sel, counts = _filter_edges(src, dst)

    for i, (din, dout) in enumerate(LAYER_DIMS):
        p = params[i]
        y = _mm(x, p["We"])
        m = _segmax[dout](y, sel, counts)
        x = _post(x, y, m, p, din, dout)

    return _pool(x, batch3)
